# baseline (speedup 1.0000x reference)
import functools

import jax
import jax.numpy as jnp
from jax.experimental import pallas as pl

_PURE = 20
_ITERS = 48


def _sparsemax_block(x_ref, o_ref):
    x = x_ref[...]
    m = jnp.max(x, axis=-1, keepdims=True)
    lo = m - 1.0
    hi = m
    mask = x > lo
    k = jnp.sum(mask.astype(jnp.float32), axis=-1, keepdims=True)
    s = jnp.sum(jnp.where(mask, x, 0.0), axis=-1, keepdims=True)

    k2 = jnp.ones_like(m)
    s2 = m

    def cond(carry):
        i, lo, hi, k, s, k2, s2 = carry
        f_lo = s - k * lo - 1.0
        eps_f = 2.5e-7 * jnp.maximum(1.0, jnp.abs(s))
        return jnp.logical_and(i < _ITERS, jnp.any(f_lo > eps_f))

    def body(carry):
        i, lo, hi, k, s, k2, s2 = carry
        nb = jnp.maximum((s - 1.0) / k, (s2 - 1.0) / k2)
        mid = 0.5 * (lo + hi)
        t = jnp.where(i < _PURE, jnp.where(nb > lo, nb, mid),
                      jnp.maximum(nb, mid))
        maskt = x > t
        kt = jnp.sum(maskt.astype(jnp.float32), axis=-1, keepdims=True)
        st = jnp.sum(jnp.where(maskt, x, 0.0), axis=-1, keepdims=True)
        ft = st - kt * t - 1.0
        good = ft >= 0.0
        lo = jnp.where(good, t, lo)
        hi = jnp.where(good, hi, t)
        k = jnp.where(good, kt, k)
        s = jnp.where(good, st, s)
        k2 = jnp.where(good, k2, kt)
        s2 = jnp.where(good, s2, st)
        return i + 1, lo, hi, k, s, k2, s2

    _, lo, hi, k, s, k2, s2 = jax.lax.while_loop(
        cond, body, (0, lo, hi, k, s, k2, s2))
    tau = jnp.maximum((s - 1.0) / k, (s2 - 1.0) / k2)
    o_ref[...] = jnp.maximum(x - tau, 0.0)


@functools.partial(jax.jit, static_argnames=("block_rows",))
def _sparsemax(x, block_rows=64):
    rows, n = x.shape
    grid = (rows // block_rows,)
    return pl.pallas_call(
        _sparsemax_block,
        grid=grid,
        in_specs=[pl.BlockSpec((block_rows, n), lambda i: (i, 0))],
        out_specs=pl.BlockSpec((block_rows, n), lambda i: (i, 0)),
        out_shape=jax.ShapeDtypeStruct((rows, n), x.dtype),
    )(x)


def kernel(x):
    return _sparsemax(x)


# speedup vs baseline: 1.2502x; 1.2502x over previous
import functools

import jax
import jax.numpy as jnp
from jax.experimental import pallas as pl

_PURE = 20
_ITERS = 48


def _sparsemax_block(x_ref, o_ref):
    x = x_ref[...]
    m = jnp.max(x, axis=-1, keepdims=True)
    lo = m - 1.0
    hi = m
    mask = x > lo
    k = jnp.sum(mask.astype(jnp.float32), axis=-1, keepdims=True)
    s = jnp.sum(jnp.where(mask, x, 0.0), axis=-1, keepdims=True)

    k2 = jnp.ones_like(m)
    s2 = m

    def cond(carry):
        i, lo, hi, k, s, k2, s2 = carry
        nb = jnp.maximum((s - 1.0) / k, (s2 - 1.0) / k2)
        f_lo = s - k * lo - 1.0
        eps_f = 1e-6 * jnp.maximum(1.0, jnp.abs(s))
        live = jnp.logical_and(nb > lo, f_lo > eps_f)
        return jnp.logical_and(i < _ITERS, jnp.any(live))

    def body(carry):
        i, lo, hi, k, s, k2, s2 = carry
        nb = jnp.maximum((s - 1.0) / k, (s2 - 1.0) / k2)
        mid = 0.5 * (lo + hi)
        t = jnp.where(i < _PURE, jnp.where(nb > lo, nb, mid),
                      jnp.maximum(nb, mid))
        maskt = x > t
        kt = jnp.sum(maskt.astype(jnp.float32), axis=-1, keepdims=True)
        st = jnp.sum(jnp.where(maskt, x, 0.0), axis=-1, keepdims=True)
        ft = st - kt * t - 1.0
        good = ft >= 0.0
        lo = jnp.where(good, t, lo)
        hi = jnp.where(good, hi, t)
        k = jnp.where(good, kt, k)
        s = jnp.where(good, st, s)
        k2 = jnp.where(good, k2, kt)
        s2 = jnp.where(good, s2, st)
        return i + 1, lo, hi, k, s, k2, s2

    _, lo, hi, k, s, k2, s2 = jax.lax.while_loop(
        cond, body, (0, lo, hi, k, s, k2, s2))
    tau = jnp.maximum((s - 1.0) / k, (s2 - 1.0) / k2)
    o_ref[...] = jnp.maximum(x - tau, 0.0)


@functools.partial(jax.jit, static_argnames=("block_rows",))
def _sparsemax(x, block_rows=8):
    rows, n = x.shape
    grid = (rows // block_rows,)
    return pl.pallas_call(
        _sparsemax_block,
        grid=grid,
        in_specs=[pl.BlockSpec((block_rows, n), lambda i: (i, 0))],
        out_specs=pl.BlockSpec((block_rows, n), lambda i: (i, 0)),
        out_shape=jax.ShapeDtypeStruct((rows, n), x.dtype),
    )(x)


def kernel(x):
    return _sparsemax(x)


# speedup vs baseline: 1.8089x; 1.4469x over previous
import functools

import jax
import jax.numpy as jnp
from jax.experimental import pallas as pl

_PURE = 20
_ITERS = 48


def _sparsemax_block(x_ref, o_ref):
    x = x_ref[...]
    m = jnp.max(x, axis=-1, keepdims=True)
    lo = m - 1.0
    hi = m
    mask = x > lo
    k = jnp.sum(mask.astype(jnp.float32), axis=-1, keepdims=True)
    s = jnp.sum(jnp.where(mask, x, 0.0), axis=-1, keepdims=True)

    k2 = jnp.ones_like(m)
    s2 = m

    def cond(carry):
        i, lo, hi, k, s, k2, s2, _ = carry
        nb = jnp.maximum((s - 1.0) / k, (s2 - 1.0) / k2)
        f_lo = s - k * lo - 1.0
        eps_f = 1e-6 * jnp.maximum(1.0, jnp.abs(s))
        live = jnp.logical_and(nb > lo, f_lo > eps_f)
        return jnp.logical_and(i < _ITERS, jnp.any(live))

    def body(carry):
        i, lo, hi, k, s, k2, s2, pt = carry
        nb = jnp.maximum((s - 1.0) / k, (s2 - 1.0) / k2)
        mid = 0.5 * (lo + hi)
        t = jnp.where(i < _PURE, jnp.where(nb > lo, nb, mid),
                      jnp.maximum(nb, mid))
        t = jnp.clip(t, lo, hi)
        t = jnp.where(t == pt, mid, t)
        maskt = x > t
        kt = jnp.sum(maskt.astype(jnp.float32), axis=-1, keepdims=True)
        st = jnp.sum(jnp.where(maskt, x, 0.0), axis=-1, keepdims=True)
        ft = st - kt * t - 1.0
        good = ft >= 0.0
        lo = jnp.where(good, t, lo)
        hi = jnp.where(good, hi, t)
        k = jnp.where(good, kt, k)
        s = jnp.where(good, st, s)
        k2 = jnp.where(good, k2, kt)
        s2 = jnp.where(good, s2, st)
        return i + 1, lo, hi, k, s, k2, s2, t

    _, lo, hi, k, s, k2, s2, _ = jax.lax.while_loop(
        cond, body, (0, lo, hi, k, s, k2, s2, hi + 1.0))
    tau = jnp.maximum((s - 1.0) / k, (s2 - 1.0) / k2)
    o_ref[...] = jnp.maximum(x - tau, 0.0)


@functools.partial(jax.jit, static_argnames=("block_rows",))
def _sparsemax(x, block_rows=8):
    rows, n = x.shape
    grid = (rows // block_rows,)
    return pl.pallas_call(
        _sparsemax_block,
        grid=grid,
        in_specs=[pl.BlockSpec((block_rows, n), lambda i: (i, 0))],
        out_specs=pl.BlockSpec((block_rows, n), lambda i: (i, 0)),
        out_shape=jax.ShapeDtypeStruct((rows, n), x.dtype),
    )(x)


def kernel(x):
    return _sparsemax(x)


# speedup vs baseline: 2.0577x; 1.1376x over previous
import functools

import jax
import jax.numpy as jnp
from jax.experimental import pallas as pl

_PURE = 20
_ITERS = 48


def _sparsemax_block(x_ref, o_ref):
    x = x_ref[...]
    m = jnp.max(x, axis=-1, keepdims=True)
    lo = m - 1.0
    hi = m
    mask = x > lo
    k = jnp.sum(mask.astype(jnp.float32), axis=-1, keepdims=True)
    s = jnp.sum(jnp.where(mask, x, 0.0), axis=-1, keepdims=True)

    k2 = jnp.ones_like(m)
    s2 = m

    def cond(carry):
        i, lo, hi, k, s, k2, s2, _ = carry
        nb = jnp.maximum((s - 1.0) / k, (s2 - 1.0) / k2)
        f_lo = s - k * lo - 1.0
        eps_f = 1e-6 * jnp.maximum(1.0, jnp.abs(s))
        live = jnp.logical_and(nb > lo, f_lo > eps_f)
        return jnp.logical_and(i < _ITERS, jnp.any(live))

    def body(carry):
        i, lo, hi, k, s, k2, s2, pt = carry
        nb = jnp.maximum((s - 1.0) / k, (s2 - 1.0) / k2)
        mid = 0.5 * (lo + hi)
        t = jnp.where(i < _PURE, jnp.where(nb > lo, nb, mid),
                      jnp.maximum(nb, mid))
        t = jnp.clip(t, lo, hi)
        t = jnp.where(t == pt, mid, t)
        maskt = x > t
        kt = jnp.sum(maskt.astype(jnp.float32), axis=-1, keepdims=True)
        st = jnp.sum(jnp.where(maskt, x, 0.0), axis=-1, keepdims=True)
        ft = st - kt * t - 1.0
        good = ft >= 0.0
        lo = jnp.where(good, t, lo)
        hi = jnp.where(good, hi, t)
        k = jnp.where(good, kt, k)
        s = jnp.where(good, st, s)
        k2 = jnp.where(good, k2, kt)
        s2 = jnp.where(good, s2, st)
        return i + 1, lo, hi, k, s, k2, s2, t

    _, lo, hi, k, s, k2, s2, _ = jax.lax.while_loop(
        cond, body, (0, lo, hi, k, s, k2, s2, hi + 1.0))
    tau = jnp.maximum((s - 1.0) / k, (s2 - 1.0) / k2)
    o_ref[...] = jnp.maximum(x - tau, 0.0)


@functools.partial(jax.jit, static_argnames=("block_rows",))
def _sparsemax(x, block_rows=16):
    rows, n = x.shape
    grid = (rows // block_rows,)
    return pl.pallas_call(
        _sparsemax_block,
        grid=grid,
        in_specs=[pl.BlockSpec((block_rows, n), lambda i: (i, 0))],
        out_specs=pl.BlockSpec((block_rows, n), lambda i: (i, 0)),
        out_shape=jax.ShapeDtypeStruct((rows, n), x.dtype),
    )(x)


def kernel(x):
    return _sparsemax(x)
